# baseline (speedup 1.0000x reference)
import functools

import jax
import jax.numpy as jnp
from jax import lax
from jax.experimental import pallas as pl
from jax.experimental.pallas import tpu as pltpu
from jax.experimental.pallas import tpu_sc as plsc

D_MODEL = 64
SCALE = 8.0
LANES = 16
IDX_W = 128
K = 4
C = K * IDX_W


@functools.cache
def _make_gather(B: int):
    info = plsc.get_sparse_core_info()
    NC, NS = info.num_cores, info.num_subcores
    NW = NC * NS
    assert B % (NW * C) == 0, (B, NW, C)
    n_chunks = B // (NW * C)
    rows_per_w = B // NW

    mesh = plsc.VectorSubcoreMesh(core_axis_name="c", subcore_axis_name="s")

    @functools.partial(
        pl.kernel,
        mesh=mesh,
        compiler_params=pltpu.CompilerParams(use_tc_tiling_on_sc=False),
        out_type=jax.ShapeDtypeStruct((B, D_MODEL), jnp.float32),
        scratch_types=[
            pltpu.VMEM((K, IDX_W), jnp.int32),
            pltpu.VMEM((C, D_MODEL), jnp.float32),
            pltpu.SemaphoreType.DMA,
        ],
    )
    def gather_kernel(x_hbm, table_hbm, out_hbm, idx_v, rows_v, sem):
        wid = lax.axis_index("s") * NC + lax.axis_index("c")
        idx_row0 = wid * (rows_per_w // IDX_W)
        out0 = wid * rows_per_w

        def chunk(n, carry):
            pltpu.sync_copy(x_hbm.at[pl.ds(idx_row0 + n * K, K)], idx_v)
            copies = [
                pltpu.async_copy(
                    table_hbm.at[idx_v.at[j]],
                    rows_v.at[pl.ds(j * IDX_W, IDX_W)],
                    sem,
                )
                for j in range(K)
            ]
            for cp in copies:
                cp.wait()

            def scale_row(r, c2):
                for v in range(D_MODEL // LANES):
                    sl = pl.ds(v * LANES, LANES)
                    rows_v[r, sl] = rows_v[r, sl] * SCALE
                return c2

            lax.fori_loop(0, C, scale_row, None)
            pltpu.sync_copy(rows_v, out_hbm.at[pl.ds(out0 + n * C, C)])
            return carry

        lax.fori_loop(0, n_chunks, chunk, None)

    return gather_kernel


def kernel(x, table):
    s0, s1 = x.shape
    b = s0 * s1
    xf = x.reshape(b // IDX_W, IDX_W).astype(jnp.int32)
    out = _make_gather(b)(xf, table)
    return out.reshape(s0, s1, D_MODEL)


# speedup vs baseline: 1.1354x; 1.1354x over previous
import functools

import jax
import jax.numpy as jnp
from jax import lax
from jax.experimental import pallas as pl
from jax.experimental.pallas import tpu as pltpu
from jax.experimental.pallas import tpu_sc as plsc

D_MODEL = 64
SCALE = 8.0
LANES = 16
IDX_W = 128
K = 5
C = K * IDX_W
UNROLL = 8


@functools.cache
def _make_gather(B: int):
    info = plsc.get_sparse_core_info()
    NC, NS = info.num_cores, info.num_subcores
    NW = NC * NS
    assert B % (NW * C) == 0, (B, NW, C)
    n_chunks = B // (NW * C)
    assert n_chunks % 2 == 0, n_chunks
    rows_per_w = B // NW
    idx_rows_per_w = rows_per_w // IDX_W

    mesh = plsc.VectorSubcoreMesh(core_axis_name="c", subcore_axis_name="s")

    @functools.partial(
        pl.kernel,
        mesh=mesh,
        compiler_params=pltpu.CompilerParams(use_tc_tiling_on_sc=False),
        out_type=jax.ShapeDtypeStruct((B, D_MODEL), jnp.float32),
        scratch_types=[
            pltpu.VMEM((idx_rows_per_w, IDX_W), jnp.int32),
            pltpu.VMEM((C, D_MODEL), jnp.float32),
            pltpu.VMEM((C, D_MODEL), jnp.float32),
            pltpu.SemaphoreType.DMA,
            pltpu.SemaphoreType.DMA,
            pltpu.SemaphoreType.DMA,
            pltpu.SemaphoreType.DMA,
        ],
    )
    def gather_kernel(x_hbm, table_hbm, out_hbm, idx_all, rows0, rows1,
                      gsem0, gsem1, osem0, osem1):
        wid = lax.axis_index("s") * NC + lax.axis_index("c")
        out0 = wid * rows_per_w
        rows = (rows0, rows1)
        gsems = (gsem0, gsem1)
        osems = (osem0, osem1)

        def start_gather(t, b):
            for j in range(K):
                pltpu.async_copy(
                    table_hbm.at[idx_all.at[t * K + j]],
                    rows[b].at[pl.ds(j * IDX_W, IDX_W)],
                    gsems[b],
                )

        def wait_gather(b):
            pltpu.make_async_copy(
                table_hbm.at[pl.ds(0, C)], rows[b], gsems[b]
            ).wait()

        def wait_out(b):
            pltpu.make_async_copy(
                rows[b], out_hbm.at[pl.ds(0, C)], osems[b]
            ).wait()

        def scale(b):
            buf = rows[b]

            def scale_grp(g, carry):
                r0 = g * UNROLL
                for rr in range(UNROLL):
                    for v in range(D_MODEL // LANES):
                        sl = pl.ds(v * LANES, LANES)
                        buf[r0 + rr, sl] = buf[r0 + rr, sl] * SCALE
                return carry

            lax.fori_loop(0, C // UNROLL, scale_grp, None)

        pltpu.sync_copy(
            x_hbm.at[pl.ds(wid * idx_rows_per_w, idx_rows_per_w)], idx_all
        )
        start_gather(0, 0)

        def outer(m2, carry):
            for b in range(2):
                t = m2 * 2 + b
                if b == 0:
                    @pl.when(m2 >= 1)
                    def _():
                        wait_out(1 - b)
                    start_gather(t + 1, 1 - b)
                else:
                    @pl.when(m2 <= n_chunks // 2 - 2)
                    def _():
                        wait_out(1 - b)
                        start_gather(t + 1, 1 - b)
                wait_gather(b)
                scale(b)
                pltpu.async_copy(
                    rows[b], out_hbm.at[pl.ds(out0 + t * C, C)], osems[b]
                )
            return carry

        lax.fori_loop(0, n_chunks // 2, outer, None)
        wait_out(0)
        wait_out(1)

    return gather_kernel


def kernel(x, table):
    s0, s1 = x.shape
    b = s0 * s1
    xf = x.reshape(b // IDX_W, IDX_W).astype(jnp.int32)
    out = _make_gather(b)(xf, table)
    return out.reshape(s0, s1, D_MODEL)
